# baseline (device time: 32410 ns/iter reference)
import jax
import jax.numpy as jnp
from jax import lax
from jax.experimental import pallas as pl
from jax.experimental.pallas import tpu as pltpu

N_DEV = 4
N_LAYERS = 3


def kernel(x, Win0, Wout0, Win1, Wout1, Win2, Wout2):
    B, D = x.shape
    H = Win0.shape[1]
    R = B // N_DEV

    def body(x_ref, win0_ref, wout0_ref, win1_ref, wout1_ref, win2_ref,
             wout2_ref, out_ref, win_b, wout_b, pa_ref, ra_ref, ps_ref,
             rb_ref, sa_sems, ra_sems, sb_sems, rb_sems):
        my = lax.axis_index("i")
        part_a = my ^ 1
        part_b = 3 - my

        for l, (wi, wo) in enumerate([(win0_ref, wout0_ref),
                                      (win1_ref, wout1_ref),
                                      (win2_ref, wout2_ref)]):
            win_b[l, :, :] = wi[:, :].astype(jnp.bfloat16)
            wout_b[l, :, :] = wo[:, :].astype(jnp.bfloat16)

        started = []

        def mlp_chunk(xc, l):
            h = jnp.maximum(
                jnp.dot(xc, win_b[l, :, :],
                        preferred_element_type=jnp.float32),
                0.0).astype(jnp.bfloat16)
            return jnp.dot(h, wout_b[l, :, :],
                           preferred_element_type=jnp.float32)

        def exchange(src, dst, send_sem, recv_sem, target):
            rdma = pltpu.make_async_remote_copy(
                src_ref=src, dst_ref=dst, send_sem=send_sem,
                recv_sem=recv_sem, device_id=(target,),
                device_id_type=pl.DeviceIdType.MESH,
            )
            rdma.start()
            started.append(rdma)

        def wait_in(dst, recv_sem, src_dummy, target):
            pltpu.make_async_remote_copy(
                src_ref=src_dummy, dst_ref=dst, send_sem=recv_sem,
                recv_sem=recv_sem, device_id=(target,),
                device_id_type=pl.DeviceIdType.MESH,
            ).wait_recv()

        def compute_send_a(l, c, xc):
            pa_ref[l, c, :, :] = mlp_chunk(xc, l).astype(jnp.bfloat16)
            exchange(pa_ref.at[l, c], ra_ref.at[l, c],
                     sa_sems.at[l, c], ra_sems.at[l, c], part_a)

        def combine_send_b(l, c):
            wait_in(ra_ref.at[l, c], ra_sems.at[l, c],
                    pa_ref.at[l, c], part_a)
            ps = pa_ref[l, c, :, :].astype(jnp.float32) + \
                 ra_ref[l, c, :, :].astype(jnp.float32)
            ps_ref[l, c, :, :] = ps.astype(jnp.bfloat16)
            exchange(ps_ref.at[l, c], rb_ref.at[l, c],
                     sb_sems.at[l, c], rb_sems.at[l, c], part_b)

        def full_chunk(l, c):
            wait_in(rb_ref.at[l, c], rb_sems.at[l, c],
                    ps_ref.at[l, c], part_b)
            return ps_ref[l, c, :, :].astype(jnp.float32) + \
                   rb_ref[l, c, :, :].astype(jnp.float32)

        for c in range(N_DEV):
            compute_send_a(0, c, x_ref[pl.ds(c * R, R), :].astype(jnp.bfloat16))
        for c in range(N_DEV):
            combine_send_b(0, c)

        for l in (1, 2):
            for c in range(N_DEV):
                compute_send_a(l, c, full_chunk(l - 1, c).astype(jnp.bfloat16))
            for c in range(N_DEV):
                combine_send_b(l, c)

        for c in range(N_DEV):
            wait_in(rb_ref.at[2, c], rb_sems.at[2, c], ps_ref.at[2, c],
                    part_b)
        out_ref[:, :] = ps_ref[2, my, :, :].astype(jnp.float32) + \
                        rb_ref[2, my, :, :].astype(jnp.float32)

        for rdma in started:
            rdma.wait_send()

    return pl.pallas_call(
        body,
        out_shape=jax.ShapeDtypeStruct((R, D), jnp.float32),
        in_specs=[pl.BlockSpec(memory_space=pltpu.VMEM)] * 7,
        out_specs=pl.BlockSpec(memory_space=pltpu.VMEM),
        scratch_shapes=[
            pltpu.VMEM((N_LAYERS, D, H), jnp.bfloat16),
            pltpu.VMEM((N_LAYERS, H, D), jnp.bfloat16),
            pltpu.VMEM((N_LAYERS, N_DEV, R, D), jnp.bfloat16),
            pltpu.VMEM((N_LAYERS, N_DEV, R, D), jnp.bfloat16),
            pltpu.VMEM((N_LAYERS, N_DEV, R, D), jnp.bfloat16),
            pltpu.VMEM((N_LAYERS, N_DEV, R, D), jnp.bfloat16),
            pltpu.SemaphoreType.DMA((N_LAYERS, N_DEV)),
            pltpu.SemaphoreType.DMA((N_LAYERS, N_DEV)),
            pltpu.SemaphoreType.DMA((N_LAYERS, N_DEV)),
            pltpu.SemaphoreType.DMA((N_LAYERS, N_DEV)),
        ],
    )(x, Win0, Wout0, Win1, Wout1, Win2, Wout2)


# device time: 29332 ns/iter; 1.1049x vs baseline; 1.1049x over previous
import jax
import jax.numpy as jnp
from jax import lax
from jax.experimental import pallas as pl
from jax.experimental.pallas import tpu as pltpu

N_DEV = 4
N_LAYERS = 3


def kernel(x, Win0, Wout0, Win1, Wout1, Win2, Wout2):
    B, D = x.shape
    H = Win0.shape[1]
    R = B // N_DEV

    def body(x_ref, win0_ref, wout0_ref, win1_ref, wout1_ref, win2_ref,
             wout2_ref, out_ref, bc_ref, part_ref, rs_ref,
             send_b, recv_b, send_rs, recv_rs):
        my = lax.axis_index("i")
        wins = [win0_ref, win1_ref, win2_ref]
        wouts = [wout0_ref, wout1_ref, wout2_ref]

        started = []

        def mlp_chunk(xc, l):
            h = jnp.maximum(
                jnp.dot(xc, wins[l][:, :],
                        preferred_element_type=jnp.float32),
                0.0)
            return jnp.dot(h, wouts[l][:, :],
                           preferred_element_type=jnp.float32)

        def bcast_chunk(l, c):
            src = bc_ref.at[l, my, pl.ds(c * R, R), :]
            for o in (2, 1, 3):
                e = (my + o) % N_DEV
                rdma = pltpu.make_async_remote_copy(
                    src_ref=src, dst_ref=src,
                    send_sem=send_b.at[l, c, o - 1],
                    recv_sem=recv_b.at[l, my, c],
                    device_id=(e,), device_id_type=pl.DeviceIdType.MESH,
                )
                rdma.start()
                started.append(rdma)

        def gather_chunk(l, c):
            acc = bc_ref[l, my, pl.ds(c * R, R), :].astype(jnp.float32)
            for o in (1, 3, 2):
                s = (my + o) % N_DEV
                pltpu.make_async_remote_copy(
                    src_ref=bc_ref.at[l, s, pl.ds(c * R, R), :],
                    dst_ref=bc_ref.at[l, s, pl.ds(c * R, R), :],
                    send_sem=send_b.at[l, c, 0],
                    recv_sem=recv_b.at[l, s, c],
                    device_id=(s,), device_id_type=pl.DeviceIdType.MESH,
                ).wait_recv()
                acc = acc + bc_ref[l, s, pl.ds(c * R, R), :].astype(jnp.float32)
            return acc

        def rs_send(c):
            return pltpu.make_async_remote_copy(
                src_ref=part_ref.at[c],
                dst_ref=rs_ref.at[my],
                send_sem=send_rs.at[c],
                recv_sem=recv_rs.at[my],
                device_id=(c,), device_id_type=pl.DeviceIdType.MESH,
            )

        for l in (0, 1):
            for c in range(N_DEV):
                if l == 0:
                    xc = x_ref[pl.ds(c * R, R), :]
                else:
                    xc = gather_chunk(0, c)
                bc_ref[l, my, pl.ds(c * R, R), :] = \
                    mlp_chunk(xc, l).astype(jnp.bfloat16)
                bcast_chunk(l, c)

        for c in range(N_DEV):
            xc = gather_chunk(1, c)
            part_ref[c, :, :] = mlp_chunk(xc, 2).astype(jnp.bfloat16)

            @pl.when(c != my)
            def _():
                rs_send(c).start()

        acc = part_ref[my, :, :].astype(jnp.float32)
        for o in (1, 3, 2):
            s = (my + o) % N_DEV
            pltpu.make_async_remote_copy(
                src_ref=rs_ref.at[s], dst_ref=rs_ref.at[s],
                send_sem=send_rs.at[0], recv_sem=recv_rs.at[s],
                device_id=(s,), device_id_type=pl.DeviceIdType.MESH,
            ).wait_recv()
            acc = acc + rs_ref[s, :, :].astype(jnp.float32)
        out_ref[:, :] = acc

        for rdma in started:
            rdma.wait_send()
        for c in range(N_DEV):
            @pl.when(c != my)
            def _():
                rs_send(c).wait_send()

    return pl.pallas_call(
        body,
        out_shape=jax.ShapeDtypeStruct((R, D), jnp.float32),
        in_specs=[pl.BlockSpec(memory_space=pltpu.VMEM)] * 7,
        out_specs=pl.BlockSpec(memory_space=pltpu.VMEM),
        scratch_shapes=[
            pltpu.VMEM((2, N_DEV, B, D), jnp.bfloat16),
            pltpu.VMEM((N_DEV, R, D), jnp.bfloat16),
            pltpu.VMEM((N_DEV, R, D), jnp.bfloat16),
            pltpu.SemaphoreType.DMA((2, N_DEV, N_DEV - 1)),
            pltpu.SemaphoreType.DMA((2, N_DEV, N_DEV)),
            pltpu.SemaphoreType.DMA((N_DEV,)),
            pltpu.SemaphoreType.DMA((N_DEV,)),
        ],
    )(x, Win0, Wout0, Win1, Wout1, Win2, Wout2)
